# Initial kernel scaffold; baseline (speedup 1.0000x reference)
import jax
import jax.numpy as jnp
from jax.experimental import pallas as pl


def kernel(inputs, codebook):
    raise NotImplementedError("write your pallas kernel here")


# speedup vs baseline: 4.5750x; 4.5750x over previous
import jax
import jax.numpy as jnp
from jax.experimental import pallas as pl
from jax.experimental.pallas import tpu as pltpu

_ROWS = 1024


def _vq_onehot_kernel(x_ref, cb_ref, out_ref):
    x = x_ref[...]
    cb = cb_ref[...]
    x2 = jnp.sum(x * x, axis=1, keepdims=True)
    e2 = jnp.sum(cb * cb, axis=1)
    prod = jax.lax.dot_general(
        x, cb, (((1,), (1,)), ((), ())),
        preferred_element_type=jnp.float32,
    )
    d = x2 + e2[None, :] - 2.0 * prod
    m = jnp.min(d, axis=1, keepdims=True)
    k = d.shape[1]
    iota = jax.lax.broadcasted_iota(jnp.int32, d.shape, 1)
    masked = jnp.where(d == m, iota, k)
    idx = jnp.min(masked, axis=1, keepdims=True)
    out_ref[...] = (iota == idx).astype(jnp.float32)


def kernel(inputs, codebook):
    n, d = inputs.shape
    k, _ = codebook.shape
    grid = (n // _ROWS,)
    encodings = pl.pallas_call(
        _vq_onehot_kernel,
        grid=grid,
        in_specs=[
            pl.BlockSpec((_ROWS, d), lambda i: (i, 0)),
            pl.BlockSpec((k, d), lambda i: (0, 0)),
        ],
        out_specs=pl.BlockSpec((_ROWS, k), lambda i: (i, 0)),
        out_shape=jax.ShapeDtypeStruct((n, k), jnp.float32),
    )(inputs, codebook)
    return (encodings, codebook)


# speedup vs baseline: 4.7231x; 1.0324x over previous
import jax
import jax.numpy as jnp
from jax.experimental import pallas as pl
from jax.experimental.pallas import tpu as pltpu

_ROWS = 1024


def _vq_onehot_kernel(x_ref, cb_ref, out_ref, e2_ref):
    @pl.when(pl.program_id(0) == 0)
    def _():
        cb0 = cb_ref[...]
        e2_ref[...] = jnp.sum(cb0 * cb0, axis=1)[None, :]

    x = x_ref[...]
    x2 = jnp.sum(x * x, axis=1, keepdims=True)
    p2 = jax.lax.dot_general(
        x + x, cb_ref[...], (((1,), (1,)), ((), ())),
        preferred_element_type=jnp.float32,
    )
    d = (x2 + e2_ref[...]) - p2
    m = jnp.min(d, axis=1, keepdims=True)
    k = d.shape[1]
    iota = jax.lax.broadcasted_iota(jnp.int32, (1, k), 1).astype(jnp.float32)
    masked = jnp.where(d == m, iota, jnp.float32(k))
    idx = jnp.min(masked, axis=1, keepdims=True)
    out_ref[...] = (iota == idx).astype(jnp.float32)


def kernel(inputs, codebook):
    n, d = inputs.shape
    k, _ = codebook.shape
    grid = (n // _ROWS,)
    encodings = pl.pallas_call(
        _vq_onehot_kernel,
        grid=grid,
        in_specs=[
            pl.BlockSpec((_ROWS, d), lambda i: (i, 0)),
            pl.BlockSpec((k, d), lambda i: (0, 0)),
        ],
        out_specs=pl.BlockSpec((_ROWS, k), lambda i: (i, 0)),
        out_shape=jax.ShapeDtypeStruct((n, k), jnp.float32),
        scratch_shapes=[pltpu.VMEM((1, k), jnp.float32)],
    )(inputs, codebook)
    return (encodings, codebook)


# speedup vs baseline: 5.2986x; 1.1218x over previous
import jax
import jax.numpy as jnp
from jax.experimental import pallas as pl
from jax.experimental.pallas import tpu as pltpu

_ROWS = 2048


def _vq_onehot_kernel(x_ref, cb_ref, out_ref, e2_ref):
    @pl.when(pl.program_id(0) == 0)
    def _():
        cb0 = cb_ref[...]
        e2_ref[...] = jnp.sum(cb0 * cb0, axis=1)[None, :]

    x = x_ref[...]
    x2 = jnp.sum(x * x, axis=1, keepdims=True)
    p2 = jax.lax.dot_general(
        x + x, cb_ref[...], (((1,), (1,)), ((), ())),
        preferred_element_type=jnp.float32,
    )
    d = (x2 + e2_ref[...]) - p2
    m = jnp.min(d, axis=1, keepdims=True)
    k = d.shape[1]
    iota = jax.lax.broadcasted_iota(jnp.int32, (1, k), 1).astype(jnp.float32)
    masked = jnp.where(d == m, iota, jnp.float32(k))
    idx = jnp.min(masked, axis=1, keepdims=True)
    out_ref[...] = (iota == idx).astype(jnp.float32)


def kernel(inputs, codebook):
    n, d = inputs.shape
    k, _ = codebook.shape
    grid = (n // _ROWS,)
    encodings = pl.pallas_call(
        _vq_onehot_kernel,
        grid=grid,
        in_specs=[
            pl.BlockSpec((_ROWS, d), lambda i: (i, 0)),
            pl.BlockSpec((k, d), lambda i: (0, 0)),
        ],
        out_specs=pl.BlockSpec((_ROWS, k), lambda i: (i, 0)),
        out_shape=jax.ShapeDtypeStruct((n, k), jnp.float32),
        scratch_shapes=[pltpu.VMEM((1, k), jnp.float32)],
    )(inputs, codebook)
    return (encodings, codebook)
